# Initial kernel scaffold; baseline (speedup 1.0000x reference)
import jax
import jax.numpy as jnp
from jax.experimental import pallas as pl


def kernel(x, index, weight):
    raise NotImplementedError("write your pallas kernel here")


# speedup vs baseline: 2.1069x; 2.1069x over previous
import functools

import jax
import jax.numpy as jnp
from jax import lax
from jax.experimental import pallas as pl
from jax.experimental.pallas import tpu as pltpu
from jax.experimental.pallas import tpu_sc as plsc

NLAT = 721
NLON = 1440
W = NLON + 1
R = NLAT * W
NQ = 786432
CH = 16

NC = 2
NS = 16
NW = NC * NS
QPT = NQ // NW
B = 512
G = 128
NG = B // G
NCHUNK = QPT // B


def _regrid_sc(table, idxT, wT):
  mesh = plsc.VectorSubcoreMesh(core_axis_name="c", subcore_axis_name="s")

  @functools.partial(
      pl.kernel,
      out_type=jax.ShapeDtypeStruct((CH, NQ), jnp.float32),
      mesh=mesh,
      compiler_params=pltpu.CompilerParams(
          needs_layout_passes=False, use_tc_tiling_on_sc=False),
      scratch_types=[
          pltpu.VMEM((4, B), jnp.int32),
          pltpu.VMEM((4, B), jnp.float32),
          pltpu.VMEM((4, B, CH), jnp.float32),
          pltpu.VMEM((CH, B), jnp.float32),
          pltpu.SemaphoreType.DMA,
          pltpu.SemaphoreType.DMA,
      ],
  )
  def k(table_hbm, idx_hbm, w_hbm, out_hbm, idx_v, w_v, rows_v, out_v,
        sem_iw, sem_g):
    wid = lax.axis_index("s") * NC + lax.axis_index("c")
    tile_base = wid * QPT
    lanes = lax.iota(jnp.int32, 16)

    def chunk(g, carry):
      base = tile_base + g * B
      cps = []
      for kk in range(4):
        cps.append(pltpu.async_copy(
            idx_hbm.at[kk, pl.ds(base, B)], idx_v.at[kk], sem_iw))
        cps.append(pltpu.async_copy(
            w_hbm.at[kk, pl.ds(base, B)], w_v.at[kk], sem_iw))
      for cp in cps:
        cp.wait()
      gps = []
      for kk in range(4):
        for j in range(NG):
          gps.append(pltpu.async_copy(
              table_hbm.at[idx_v.at[kk, pl.ds(j * G, G)]],
              rows_v.at[kk, pl.ds(j * G, G)], sem_g))
      for gp in gps:
        gp.wait()

      def group(gi, c):
        qb = gi * 16
        qi = qb + lanes
        w0 = w_v[0, pl.ds(qb, 16)]
        w1 = w_v[1, pl.ds(qb, 16)]
        w2 = w_v[2, pl.ds(qb, 16)]
        w3 = w_v[3, pl.ds(qb, 16)]
        for ch in range(CH):
          cs = jnp.full((16,), ch, jnp.int32)
          g0 = plsc.load_gather(rows_v.at[0], [qi, cs])
          g1 = plsc.load_gather(rows_v.at[1], [qi, cs])
          g2 = plsc.load_gather(rows_v.at[2], [qi, cs])
          g3 = plsc.load_gather(rows_v.at[3], [qi, cs])
          out_v[ch, pl.ds(qb, 16)] = g0 * w0 + g1 * w1 + g2 * w2 + g3 * w3
        return c

      lax.fori_loop(0, B // 16, group, 0)
      pltpu.sync_copy(out_v, out_hbm.at[:, pl.ds(base, B)])
      return carry

    lax.fori_loop(0, NCHUNK, chunk, 0)

  return k(table, idxT, wT)


def kernel(x, index, weight):
  xt = jnp.transpose(x[0], (1, 2, 0))
  table = jnp.concatenate([xt, xt[:, :1, :]], axis=1).reshape(R, CH)
  idxT = jnp.transpose(index, (1, 0))
  wT = jnp.transpose(weight, (1, 0))
  out = _regrid_sc(table, idxT, wT)
  return out.reshape(1, CH, NQ)
